# baseline (speedup 1.0000x reference)
import functools

import jax
import jax.numpy as jnp
from jax import lax
from jax.experimental import pallas as pl
from jax.experimental.pallas import tpu as pltpu
from jax.experimental.pallas import tpu_sc as plsc

_NEG = -3.4e38
_IMAX = 2**31 - 1

_Q = 1024
_D = 64
_N = 100000
_NPAD = 102400
_T = 3200
_NB = _NPAD // 32
_BT = _T // 32
_K = 100
_QB = 128
_NW = 32
_BPW = (_Q * _K) // _NW


def _scores_body(q_ref, b_ref, w_ref, s_ref, m_ref):
    i = pl.program_id(0)
    q = q_ref[...]
    w = w_ref[...]
    b = b_ref[...]
    qw = lax.dot_general(q, w, (((1,), (1,)), ((), ())),
                         preferred_element_type=jnp.float32)
    s = lax.dot_general(qw, b, (((1,), (1,)), ((), ())),
                        preferred_element_type=jnp.float32)
    col = i * _T + lax.broadcasted_iota(jnp.int32, (_Q, _T), 1)
    s_ref[...] = jnp.where(col < _N, s, _NEG)
    st = lax.dot_general(b, qw, (((1,), (1,)), ((), ())),
                         preferred_element_type=jnp.float32)
    row = i * _T + lax.broadcasted_iota(jnp.int32, (_T, _Q), 0)
    st = jnp.where(row < _N, st, _NEG)
    m_ref[...] = jnp.max(st.reshape(_BT, 32, _Q), axis=1).reshape(1, _BT, _Q)


def _topk_lane_body(v_ref, id_ref, o_ref):
    v0 = v_ref[...]
    ids = id_ref[...]
    tid = lax.broadcasted_iota(jnp.int32, (_QB, _QB), 1)

    def step(t, carry):
        v, acc = carry
        m = jnp.max(v, axis=1, keepdims=True)
        gid = jnp.min(jnp.where(v == m, ids, _IMAX), axis=1, keepdims=True)
        acc = jnp.where(tid == t, gid, acc)
        v = jnp.where(ids == gid, _NEG, v)
        return v, acc

    _, acc = lax.fori_loop(0, _K, step, (v0, jnp.zeros((_QB, _QB), jnp.int32)))
    o_ref[...] = acc


def _run_topk_lane(vals, ids, width):
    return pl.pallas_call(
        _topk_lane_body,
        grid=(_Q // _QB,),
        in_specs=[
            pl.BlockSpec((_QB, width), lambda i: (i, 0)),
            pl.BlockSpec((_QB, width), lambda i: (i, 0)),
        ],
        out_specs=pl.BlockSpec((_QB, _QB), lambda i: (i, 0)),
        out_shape=jax.ShapeDtypeStruct((_Q, _QB), jnp.int32),
    )(vals, ids)


def _radix_tau_body(v_ref, o_ref):
    v = v_ref[...] + 0.0
    s = lax.bitcast_convert_type(v, jnp.int32)
    key = jnp.where(s < 0, s ^ jnp.int32(0x7FFFFFFF), s)

    def pass_fn(t, p):
        b = 31 - t
        bit = jnp.left_shift(jnp.int32(1), b)
        qu = p | bit
        qs = qu ^ jnp.int32(-2147483648)
        cnt = jnp.sum((key >= qs).astype(jnp.int32), axis=1, keepdims=True)
        return jnp.where(cnt >= _K, qu, p)

    p = lax.fori_loop(0, 32, pass_fn, jnp.zeros((_QB, 1), jnp.int32))
    ks = p ^ jnp.int32(-2147483648)
    sf = jnp.where(ks < 0, ks ^ jnp.int32(0x7FFFFFFF), ks)
    o_ref[...] = jnp.broadcast_to(
        lax.bitcast_convert_type(sf, jnp.float32), (_QB, _QB))


_sc_mesh = plsc.VectorSubcoreMesh(core_axis_name="c", subcore_axis_name="s")


@functools.partial(
    pl.kernel,
    mesh=_sc_mesh,
    compiler_params=pltpu.CompilerParams(use_tc_tiling_on_sc=False),
    out_type=jax.ShapeDtypeStruct((_Q * _K, 32), jnp.float32),
    scratch_types=[
        pltpu.VMEM((_BPW,), jnp.int32),
        pltpu.VMEM((_BPW, 32), jnp.float32),
        pltpu.SemaphoreType.DMA,
    ],
)
def _gather_sc(table_hbm, idx_hbm, out_hbm, idx_v, rows_v, sem):
    wid = lax.axis_index("s") * 2 + lax.axis_index("c")
    base = wid * _BPW
    pltpu.sync_copy(idx_hbm.at[pl.ds(base, _BPW)], idx_v)
    pltpu.async_copy(table_hbm.at[idx_v], rows_v, sem).wait()
    pltpu.sync_copy(rows_v, out_hbm.at[pl.ds(base, _BPW)])


_QPW = _Q // _NW
_CPW = _QPW * _K * 32


@functools.partial(
    pl.kernel,
    mesh=_sc_mesh,
    compiler_params=pltpu.CompilerParams(use_tc_tiling_on_sc=False,
                                         needs_layout_passes=False),
    out_type=[
        jax.ShapeDtypeStruct((_Q * 256,), jnp.float32),
        jax.ShapeDtypeStruct((_Q * 256,), jnp.int32),
    ],
    scratch_types=[
        pltpu.VMEM((_CPW // 2,), jnp.float32),
        pltpu.VMEM((_CPW // 2,), jnp.int32),
        pltpu.VMEM((_QPW * 128,), jnp.float32),
        pltpu.VMEM((_QPW * 256,), jnp.float32),
        pltpu.VMEM((_QPW * 256,), jnp.int32),
    ],
)
def _compact_sc(cand_hbm, gid_hbm, tau_hbm, outv_hbm, outg_hbm,
                candf, gidsf, taurepv, ov, og):
    wid = lax.axis_index("s") * 2 + lax.axis_index("c")
    qbase = wid * _QPW
    pltpu.sync_copy(tau_hbm.at[pl.ds(qbase * 128, _QPW * 128)], taurepv)

    negv = jnp.full((16,), _NEG, jnp.float32)
    imaxv = jnp.full((16,), _IMAX, jnp.int32)

    def init_step(i, _):
        ov[pl.ds(i * 16, 16)] = negv
        og[pl.ds(i * 16, 16)] = imaxv
        return 0

    lax.fori_loop(0, _QPW * 16, init_step, 0)

    for hb in (0, 1):
        src = (qbase + hb * (_QPW // 2)) * _K * 32
        pltpu.sync_copy(cand_hbm.at[pl.ds(src, _CPW // 2)], candf)
        pltpu.sync_copy(gid_hbm.at[pl.ds(src, _CPW // 2)], gidsf)

        def per_q(qq2, _):
            qq = hb * (_QPW // 2) + qq2
            tq = taurepv[pl.ds(qq * 128, 16)]
            obase = qq * 256

            def per_r(r, carry):
                cs, ct = carry
                for h in (0, 1):
                    off = (qq2 * _K + r) * 32 + h * 16
                    v = candf[pl.ds(off, 16)]
                    gid = gidsf[pl.ds(off, 16)]
                    ms = v > tq
                    mt = v == tq
                    ns = jnp.sum(ms.astype(jnp.int32), axis=0)
                    nt = jnp.sum(mt.astype(jnp.int32), axis=0)
                    i16 = lax.iota(jnp.int32, 16)
                    ks = jnp.where(ms, i16, i16 + 16)
                    kt = jnp.where(mt, i16, i16 + 16)
                    ov[pl.ds(obase + cs, 16)] = plsc.sort_key_val(ks, v)[1]
                    og[pl.ds(obase + cs, 16)] = plsc.sort_key_val(ks, gid)[1]
                    ov[pl.ds(obase + 128 + ct, 16)] = plsc.sort_key_val(kt, v)[1]
                    og[pl.ds(obase + 128 + ct, 16)] = plsc.sort_key_val(kt, gid)[1]
                    cs = jnp.minimum(cs + ns, 112)
                    ct = jnp.minimum(ct + nt, 112)
                return cs, ct

            cs_f, ct_f = lax.fori_loop(0, _K, per_r,
                                       (jnp.int32(0), jnp.int32(0)))
            ov[pl.ds(obase + cs_f, 16)] = negv
            og[pl.ds(obase + cs_f, 16)] = imaxv
            ov[pl.ds(obase + 128 + ct_f, 16)] = negv
            og[pl.ds(obase + 128 + ct_f, 16)] = imaxv
            return 0

        lax.fori_loop(0, _QPW // 2, per_q, 0)

    pltpu.sync_copy(ov, outv_hbm.at[pl.ds(qbase * 256, _QPW * 256)])
    pltpu.sync_copy(og, outg_hbm.at[pl.ds(qbase * 256, _QPW * 256)])


def kernel(query, bank, k, W):
    del k
    bank_pad = jnp.pad(bank, ((0, _NPAD - _N), (0, 0)))
    scores, m_t = pl.pallas_call(
        _scores_body,
        grid=(_NPAD // _T,),
        in_specs=[
            pl.BlockSpec((_Q, _D), lambda i: (0, 0)),
            pl.BlockSpec((_T, _D), lambda i: (i, 0)),
            pl.BlockSpec((_D, _D), lambda i: (0, 0)),
        ],
        out_specs=[
            pl.BlockSpec((_Q, _T), lambda i: (0, i)),
            pl.BlockSpec((1, _BT, _Q), lambda i: (i, 0, 0)),
        ],
        out_shape=[
            jax.ShapeDtypeStruct((_Q, _NPAD), jnp.float32),
            jax.ShapeDtypeStruct((_NPAD // _T, _BT, _Q), jnp.float32),
        ],
    )(query, bank_pad, W)

    m = m_t.reshape(_NB, _Q).T
    bid_iota = jnp.broadcast_to(jnp.arange(_NB, dtype=jnp.int32)[None, :],
                                (_Q, _NB))
    bidx = _run_topk_lane(m, bid_iota, _NB)[:, :_K]

    table = scores.reshape(_Q * _NB, 32)
    flat_idx = (jnp.arange(_Q, dtype=jnp.int32)[:, None] * _NB
                + bidx).reshape(_Q * _K)
    cand = _gather_sc(table, flat_idx)

    vals = cand.reshape(_Q, _K * 32)
    tau = pl.pallas_call(
        _radix_tau_body,
        grid=(_Q // _QB,),
        in_specs=[pl.BlockSpec((_QB, _K * 32), lambda i: (i, 0))],
        out_specs=pl.BlockSpec((_QB, _QB), lambda i: (i, 0)),
        out_shape=jax.ShapeDtypeStruct((_Q, _QB), jnp.float32),
    )(vals)

    gids = (bidx[:, :, None] * 32
            + jnp.arange(32, dtype=jnp.int32)[None, None, :]).reshape(-1)
    sv_flat, sg_flat = _compact_sc(
        cand.reshape(_Q * _K * 32), gids, tau.reshape(_Q * 128))
    out = _run_topk_lane(sv_flat.reshape(_Q, 256), sg_flat.reshape(_Q, 256), 256)
    return out[:, :_K]


# speedup vs baseline: 1.1218x; 1.1218x over previous
import functools

import jax
import jax.numpy as jnp
from jax import lax
from jax.experimental import pallas as pl
from jax.experimental.pallas import tpu as pltpu
from jax.experimental.pallas import tpu_sc as plsc

_NEG = -3.4e38
_IMAX = 2**31 - 1

_Q = 1024
_D = 64
_N = 100000
_NPAD = 102400
_T = 3200
_NB = _NPAD // 32
_BT = _T // 32
_K = 100
_QB = 128
_NW = 32
_BPW = (_Q * _K) // _NW


def _scores_body(q_ref, b_ref, w_ref, s_ref, m_ref):
    i = pl.program_id(0)
    q = q_ref[...]
    w = w_ref[...]
    b = b_ref[...]
    qw = lax.dot_general(q, w, (((1,), (1,)), ((), ())),
                         preferred_element_type=jnp.float32)
    s = lax.dot_general(qw, b, (((1,), (1,)), ((), ())),
                        preferred_element_type=jnp.float32)
    col = i * _T + lax.broadcasted_iota(jnp.int32, (_Q, _T), 1)
    s_ref[...] = jnp.where(col < _N, s, _NEG)
    st = lax.dot_general(b, qw, (((1,), (1,)), ((), ())),
                         preferred_element_type=jnp.float32)
    row = i * _T + lax.broadcasted_iota(jnp.int32, (_T, _Q), 0)
    st = jnp.where(row < _N, st, _NEG)
    m_ref[...] = jnp.max(st.reshape(_BT, 32, _Q), axis=1).reshape(1, _BT, _Q)


def _topk_lane_body(v_ref, id_ref, o_ref):
    v0 = v_ref[...]
    ids = id_ref[...]
    tid = lax.broadcasted_iota(jnp.int32, (_QB, _QB), 1)

    def step(t, carry):
        v, acc = carry
        m = jnp.max(v, axis=1, keepdims=True)
        gid = jnp.min(jnp.where(v == m, ids, _IMAX), axis=1, keepdims=True)
        acc = jnp.where(tid == t, gid, acc)
        v = jnp.where(ids == gid, _NEG, v)
        return v, acc

    _, acc = lax.fori_loop(0, _K, step, (v0, jnp.zeros((_QB, _QB), jnp.int32)))
    o_ref[...] = acc


def _run_topk_lane(vals, ids, width):
    return pl.pallas_call(
        _topk_lane_body,
        grid=(_Q // _QB,),
        in_specs=[
            pl.BlockSpec((_QB, width), lambda i: (i, 0)),
            pl.BlockSpec((_QB, width), lambda i: (i, 0)),
        ],
        out_specs=pl.BlockSpec((_QB, _QB), lambda i: (i, 0)),
        out_shape=jax.ShapeDtypeStruct((_Q, _QB), jnp.int32),
    )(vals, ids)


def _radix_tau_body(v_ref, o_ref):
    v = v_ref[...] + 0.0
    s = lax.bitcast_convert_type(v, jnp.int32)
    key = jnp.where(s < 0, s ^ jnp.int32(0x7FFFFFFF), s)

    def pass_fn(t, p):
        b = 31 - t
        bit = jnp.left_shift(jnp.int32(1), b)
        qu = p | bit
        qs = qu ^ jnp.int32(-2147483648)
        cnt = jnp.sum((key >= qs).astype(jnp.int32), axis=1, keepdims=True)
        return jnp.where(cnt >= _K, qu, p)

    p = lax.fori_loop(0, 32, pass_fn, jnp.zeros((_QB, 1), jnp.int32))
    ks = p ^ jnp.int32(-2147483648)
    sf = jnp.where(ks < 0, ks ^ jnp.int32(0x7FFFFFFF), ks)
    o_ref[...] = jnp.broadcast_to(
        lax.bitcast_convert_type(sf, jnp.float32), (_QB, _QB))


_sc_mesh = plsc.VectorSubcoreMesh(core_axis_name="c", subcore_axis_name="s")


@functools.partial(
    pl.kernel,
    mesh=_sc_mesh,
    compiler_params=pltpu.CompilerParams(use_tc_tiling_on_sc=False),
    out_type=jax.ShapeDtypeStruct((_Q * _K, 32), jnp.float32),
    scratch_types=[
        pltpu.VMEM((_BPW,), jnp.int32),
        pltpu.VMEM((_BPW, 32), jnp.float32),
        pltpu.SemaphoreType.DMA,
    ],
)
def _gather_sc(table_hbm, idx_hbm, out_hbm, idx_v, rows_v, sem):
    wid = lax.axis_index("s") * 2 + lax.axis_index("c")
    base = wid * _BPW
    pltpu.sync_copy(idx_hbm.at[pl.ds(base, _BPW)], idx_v)
    pltpu.async_copy(table_hbm.at[idx_v], rows_v, sem).wait()
    pltpu.sync_copy(rows_v, out_hbm.at[pl.ds(base, _BPW)])


_QPW = _Q // _NW
_CPW = _QPW * _K * 32


@functools.partial(
    pl.kernel,
    mesh=_sc_mesh,
    compiler_params=pltpu.CompilerParams(use_tc_tiling_on_sc=False,
                                         needs_layout_passes=False),
    out_type=[
        jax.ShapeDtypeStruct((_Q * 256,), jnp.float32),
        jax.ShapeDtypeStruct((_Q * 256,), jnp.int32),
    ],
    scratch_types=[
        pltpu.VMEM((_CPW // 2,), jnp.float32),
        pltpu.VMEM((_CPW // 2,), jnp.int32),
        pltpu.VMEM((_QPW * 128,), jnp.float32),
        pltpu.VMEM((_QPW * 256,), jnp.float32),
        pltpu.VMEM((_QPW * 256,), jnp.int32),
    ],
)
def _compact_sc(cand_hbm, gid_hbm, tau_hbm, outv_hbm, outg_hbm,
                candf, gidsf, taurepv, ov, og):
    wid = lax.axis_index("s") * 2 + lax.axis_index("c")
    qbase = wid * _QPW
    pltpu.sync_copy(tau_hbm.at[pl.ds(qbase * 128, _QPW * 128)], taurepv)

    negv = jnp.full((16,), _NEG, jnp.float32)
    imaxv = jnp.full((16,), _IMAX, jnp.int32)

    def init_step(i, _):
        ov[pl.ds(i * 16, 16)] = negv
        og[pl.ds(i * 16, 16)] = imaxv
        return 0

    lax.fori_loop(0, _QPW * 16, init_step, 0)

    for hb in (0, 1):
        src = (qbase + hb * (_QPW // 2)) * _K * 32
        pltpu.sync_copy(cand_hbm.at[pl.ds(src, _CPW // 2)], candf)
        pltpu.sync_copy(gid_hbm.at[pl.ds(src, _CPW // 2)], gidsf)

        def per_q(qq2, _):
            qq = hb * (_QPW // 2) + qq2
            tq = taurepv[pl.ds(qq * 128, 16)]
            obase = qq * 256

            def per_r(r, carry):
                cs, ct = carry
                for h in (0, 1):
                    off = (qq2 * _K + r) * 32 + h * 16
                    v = candf[pl.ds(off, 16)]
                    gid = gidsf[pl.ds(off, 16)]
                    ms = v > tq
                    mt = v == tq
                    ns = jnp.sum(ms.astype(jnp.int32), axis=0)
                    nt = jnp.sum(mt.astype(jnp.int32), axis=0)
                    i16 = lax.iota(jnp.int32, 16)
                    ks = jnp.where(ms, i16, i16 + 16)
                    kt = jnp.where(mt, i16, i16 + 16)
                    ov[pl.ds(obase + cs, 16)] = plsc.sort_key_val(ks, v)[1]
                    og[pl.ds(obase + cs, 16)] = plsc.sort_key_val(ks, gid)[1]
                    ov[pl.ds(obase + 128 + ct, 16)] = plsc.sort_key_val(kt, v)[1]
                    og[pl.ds(obase + 128 + ct, 16)] = plsc.sort_key_val(kt, gid)[1]
                    cs = jnp.minimum(cs + ns, 112)
                    ct = jnp.minimum(ct + nt, 112)
                return cs, ct

            cs_f, ct_f = lax.fori_loop(0, _K, per_r,
                                       (jnp.int32(0), jnp.int32(0)))
            ov[pl.ds(obase + cs_f, 16)] = negv
            og[pl.ds(obase + cs_f, 16)] = imaxv
            ov[pl.ds(obase + 128 + ct_f, 16)] = negv
            og[pl.ds(obase + 128 + ct_f, 16)] = imaxv
            return 0

        lax.fori_loop(0, _QPW // 2, per_q, 0)

    pltpu.sync_copy(ov, outv_hbm.at[pl.ds(qbase * 256, _QPW * 256)])
    pltpu.sync_copy(og, outg_hbm.at[pl.ds(qbase * 256, _QPW * 256)])


def kernel(query, bank, k, W):
    del k
    bank_pad = jnp.pad(bank, ((0, _NPAD - _N), (0, 0)))
    scores, m_t = pl.pallas_call(
        _scores_body,
        grid=(_NPAD // _T,),
        in_specs=[
            pl.BlockSpec((_Q, _D), lambda i: (0, 0)),
            pl.BlockSpec((_T, _D), lambda i: (i, 0)),
            pl.BlockSpec((_D, _D), lambda i: (0, 0)),
        ],
        out_specs=[
            pl.BlockSpec((_Q, _T), lambda i: (0, i)),
            pl.BlockSpec((1, _BT, _Q), lambda i: (i, 0, 0)),
        ],
        out_shape=[
            jax.ShapeDtypeStruct((_Q, _NPAD), jnp.float32),
            jax.ShapeDtypeStruct((_NPAD // _T, _BT, _Q), jnp.float32),
        ],
    )(query, bank_pad, W)

    m = m_t.reshape(_NB, _Q).T
    tau_b = pl.pallas_call(
        _radix_tau_body,
        grid=(_Q // _QB,),
        in_specs=[pl.BlockSpec((_QB, _NB), lambda i: (i, 0))],
        out_specs=pl.BlockSpec((_QB, _QB), lambda i: (i, 0)),
        out_shape=jax.ShapeDtypeStruct((_Q, _QB), jnp.float32),
    )(m)
    bid_iota = jnp.broadcast_to(jnp.arange(_NB, dtype=jnp.int32)[None, :],
                                (_Q, _NB)).reshape(-1)
    mv_flat, mg_flat = _compact_sc(m.reshape(-1), bid_iota,
                                   tau_b.reshape(-1))
    bidx = _run_topk_lane(mv_flat.reshape(_Q, 256),
                          mg_flat.reshape(_Q, 256), 256)[:, :_K]

    table = scores.reshape(_Q * _NB, 32)
    flat_idx = (jnp.arange(_Q, dtype=jnp.int32)[:, None] * _NB
                + bidx).reshape(_Q * _K)
    cand = _gather_sc(table, flat_idx)

    vals = cand.reshape(_Q, _K * 32)
    tau = pl.pallas_call(
        _radix_tau_body,
        grid=(_Q // _QB,),
        in_specs=[pl.BlockSpec((_QB, _K * 32), lambda i: (i, 0))],
        out_specs=pl.BlockSpec((_QB, _QB), lambda i: (i, 0)),
        out_shape=jax.ShapeDtypeStruct((_Q, _QB), jnp.float32),
    )(vals)

    gids = (bidx[:, :, None] * 32
            + jnp.arange(32, dtype=jnp.int32)[None, None, :]).reshape(-1)
    sv_flat, sg_flat = _compact_sc(
        cand.reshape(_Q * _K * 32), gids, tau.reshape(_Q * 128))
    out = _run_topk_lane(sv_flat.reshape(_Q, 256), sg_flat.reshape(_Q, 256), 256)
    return out[:, :_K]
